# Initial kernel scaffold; baseline (speedup 1.0000x reference)
import jax
import jax.numpy as jnp
from jax.experimental import pallas as pl


def kernel(x, edge_index, edge_weight, W, b):
    raise NotImplementedError("write your pallas kernel here")


# speedup vs baseline: 5.3952x; 5.3952x over previous
import functools

import jax
import jax.numpy as jnp
from jax import lax
from jax.experimental import pallas as pl
from jax.experimental.pallas import tpu as pltpu
from jax.experimental.pallas import tpu_sc as plsc

N = 10000
E = 320000
D = 128

CH = 128
NCHUNK = E // CH
NW = 32
BASE_CH = NCHUNK // NW
EXTRA = NCHUNK % NW
NGROUP = N // 8
G_BASE = NGROUP // 16
G_EXTRA = NGROUP % 16
DRAIN = 104


def _linear_body(x_ref, w_ref, b_ref, o_ref):
    o_ref[...] = lax.dot_general(
        x_ref[...], w_ref[...], (((1,), (1,)), ((), ())),
        preferred_element_type=jnp.float32) + b_ref[...]


def _linear(x, W, b):
    blk = 1000
    return pl.pallas_call(
        _linear_body,
        grid=(N // blk,),
        in_specs=[
            pl.BlockSpec((blk, D), lambda i: (i, 0)),
            pl.BlockSpec((D, D), lambda i: (0, 0)),
            pl.BlockSpec((1, D), lambda i: (0, 0)),
        ],
        out_specs=pl.BlockSpec((blk, D), lambda i: (i, 0)),
        out_shape=jax.ShapeDtypeStruct((N, D), jnp.float32),
    )(x, W, b.reshape(1, D))


_SPLAT_DNUMS = lax.GatherDimensionNumbers(
    offset_dims=(), collapsed_slice_dims=(0,), start_index_map=(0,))


def _splat(vec16, lane):
    idx = jnp.full((16, 1), lane, jnp.int32)
    return lax.gather(vec16, idx, _SPLAT_DNUMS, slice_sizes=(1,),
                      mode=lax.GatherScatterMode.PROMISE_IN_BOUNDS)

def _spmm_body(h_hbm, dst_hbm, src_hbm, ew_hbm, out_hbm,
               src_v, dst_v, ew_v, rows_v, acc_sh, sem):
    cid = lax.axis_index("c")
    sid = lax.axis_index("s")
    wid = sid * 2 + cid
    row0 = pl.multiple_of(8 * (G_BASE * sid + jnp.minimum(sid, G_EXTRA)), 8)
    has_extra = sid < G_EXTRA

    zeros16 = jnp.zeros((16,), jnp.float32)

    def _zrow(i, carry):
        for j in range(8):
            rows_v[i, pl.ds(j * 16, 16)] = zeros16
        return carry

    lax.fori_loop(0, DRAIN, _zrow, 0)
    for r in range(G_BASE * 8 // DRAIN):
        pltpu.sync_copy(rows_v.at[pl.ds(0, DRAIN)],
                        acc_sh.at[pl.ds(pl.multiple_of(row0 + r * DRAIN, 8),
                                        DRAIN)])

    @pl.when(has_extra)
    def _():
        pltpu.sync_copy(
            rows_v.at[pl.ds(0, 8)],
            acc_sh.at[pl.ds(pl.multiple_of(row0 + G_BASE * 8, 8), 8)])

    plsc.subcore_barrier()

    nch = BASE_CH + jnp.where(wid < EXTRA, 1, 0)

    def _chunk(k, carry):
        base = pl.multiple_of((k * NW + wid) * CH, CH)
        pltpu.sync_copy(src_hbm.at[pl.ds(base, CH)], src_v)
        pltpu.sync_copy(dst_hbm.at[pl.ds(base, CH)], dst_v)
        pltpu.sync_copy(ew_hbm.at[pl.ds(base, CH)], ew_v)
        pltpu.async_copy(h_hbm.at[src_v], rows_v, sem).wait()

        def _scale(g, c2):
            ew16 = ew_v[pl.ds(pl.multiple_of(g * 16, 16), 16)]
            for lane in range(16):
                e = g * 16 + lane
                wv = _splat(ew16, lane)
                for j in range(8):
                    sl = pl.ds(j * 16, 16)
                    rows_v[e, sl] = rows_v[e, sl] * wv
            return c2

        lax.fori_loop(0, CH // 16, _scale, 0)
        pltpu.sync_copy(rows_v, acc_sh.at[dst_v], add=True)
        return carry

    lax.fori_loop(0, nch, _chunk, 0)
    plsc.subcore_barrier()

    for r in range(G_BASE * 8 // DRAIN):
        sl = pl.ds(pl.multiple_of(row0 + r * DRAIN, 8), DRAIN)
        pltpu.sync_copy(acc_sh.at[sl], rows_v.at[pl.ds(0, DRAIN)])
        pltpu.sync_copy(rows_v.at[pl.ds(0, DRAIN)], out_hbm.at[cid, sl])

    @pl.when(has_extra)
    def _():
        sl = pl.ds(pl.multiple_of(row0 + G_BASE * 8, 8), 8)
        pltpu.sync_copy(acc_sh.at[sl], rows_v.at[pl.ds(0, 8)])
        pltpu.sync_copy(rows_v.at[pl.ds(0, 8)], out_hbm.at[cid, sl])


def _spmm_sc(h, dst, src, ew):
    mesh = plsc.VectorSubcoreMesh(core_axis_name="c", subcore_axis_name="s")
    f = functools.partial(
        pl.kernel,
        out_type=jax.ShapeDtypeStruct((2, N, D), jnp.float32),
        mesh=mesh,
        scratch_types=[
            pltpu.VMEM((CH,), jnp.int32),
            pltpu.VMEM((CH,), jnp.int32),
            pltpu.VMEM((CH,), jnp.float32),
            pltpu.VMEM((CH, D), jnp.float32),
            pltpu.VMEM_SHARED((N, D), jnp.float32),
            pltpu.SemaphoreType.DMA,
        ],
    )(_spmm_body)
    return f(h, dst, src, ew)


def _comb_body(p_ref, o_ref):
    o_ref[...] = p_ref[0] + p_ref[1]


def _combine(parts):
    blk = 1000
    return pl.pallas_call(
        _comb_body,
        grid=(N // blk,),
        in_specs=[pl.BlockSpec((2, blk, D), lambda i: (0, i, 0))],
        out_specs=pl.BlockSpec((blk, D), lambda i: (i, 0)),
        out_shape=jax.ShapeDtypeStruct((N, D), jnp.float32),
    )(parts)


def kernel(x, edge_index, edge_weight, W, b):
    h = _linear(x, W, b)
    parts = _spmm_sc(h, edge_index[0], edge_index[1], edge_weight)
    return _combine(parts)


# speedup vs baseline: 10.9785x; 2.0349x over previous
import functools

import jax
import jax.numpy as jnp
from jax import lax
from jax.experimental import pallas as pl
from jax.experimental.pallas import tpu as pltpu
from jax.experimental.pallas import tpu_sc as plsc

N = 10000
E = 320000
D = 128

CH = 64
NCHUNK = E // CH
NW = 32
NSLOT = 4
NQUAD = NCHUNK // NSLOT
Q_EXTRA = NQUAD % NW
NGROUP = N // 8
G_BASE = NGROUP // 16
G_EXTRA = NGROUP % 16
DRAIN = 48


def _linear_body(x_ref, w_ref, b_ref, o_ref):
    o_ref[...] = lax.dot_general(
        x_ref[...], w_ref[...], (((1,), (1,)), ((), ())),
        preferred_element_type=jnp.float32) + b_ref[...]


def _linear(x, W, b):
    blk = 1000
    return pl.pallas_call(
        _linear_body,
        grid=(N // blk,),
        in_specs=[
            pl.BlockSpec((blk, D), lambda i: (i, 0)),
            pl.BlockSpec((D, D), lambda i: (0, 0)),
            pl.BlockSpec((1, D), lambda i: (0, 0)),
        ],
        out_specs=pl.BlockSpec((blk, D), lambda i: (i, 0)),
        out_shape=jax.ShapeDtypeStruct((N, D), jnp.float32),
    )(x, W, b.reshape(1, D))


_SPLAT_DNUMS = lax.GatherDimensionNumbers(
    offset_dims=(), collapsed_slice_dims=(0,), start_index_map=(0,))


def _splat(vec16, lane):
    idx = jnp.full((16, 1), lane, jnp.int32)
    return lax.gather(vec16, idx, _SPLAT_DNUMS, slice_sizes=(1,),
                      mode=lax.GatherScatterMode.PROMISE_IN_BOUNDS)


def _spmm_body(h_hbm, dst_hbm, src_hbm, ew_hbm, out_hbm,
               dst_s, src_s, ew_s, rows_s, didx, acc_sh,
               esems, gsems, ssems):
    cid = lax.axis_index("c")
    sid = lax.axis_index("s")
    wid = sid * 2 + cid
    row0 = pl.multiple_of(8 * (G_BASE * sid + jnp.minimum(sid, G_EXTRA)), 8)
    has_extra = sid < G_EXTRA

    def _edata_descs(quad, s):
        base = pl.multiple_of((quad * NSLOT + s) * CH, CH)
        return (
            pltpu.make_async_copy(dst_hbm.at[pl.ds(base, CH)], dst_s[s],
                                  esems[s]),
            pltpu.make_async_copy(src_hbm.at[pl.ds(base, CH)], src_s[s],
                                  esems[s]),
            pltpu.make_async_copy(ew_hbm.at[pl.ds(base, CH)], ew_s[s],
                                  esems[s]),
        )

    def _gather_desc(s):
        return pltpu.make_async_copy(h_hbm.at[src_s[s]], rows_s[s], gsems[s])

    def _scatter_desc(s):
        return pltpu.make_async_copy(rows_s[s], acc_sh.at[didx.at[s]],
                                     ssems[s])

    zeros16 = jnp.zeros((16,), jnp.float32)

    def _zrow(i, carry):
        for j in range(D // 16):
            rows_s[0][i, pl.ds(j * 16, 16)] = zeros16
        return carry

    lax.fori_loop(0, DRAIN, _zrow, 0)
    for r in range(G_BASE * 8 // DRAIN):
        pltpu.sync_copy(rows_s[0].at[pl.ds(0, DRAIN)],
                        acc_sh.at[pl.ds(pl.multiple_of(row0 + r * DRAIN, 8),
                                        DRAIN)])

    @pl.when(has_extra)
    def _():
        pltpu.sync_copy(
            rows_s[0].at[pl.ds(0, 8)],
            acc_sh.at[pl.ds(pl.multiple_of(row0 + G_BASE * 8, 8), 8)])

    plsc.subcore_barrier()

    nq = (NQUAD // NW) + jnp.where(wid < Q_EXTRA, 1, 0)

    @pl.when(nq > 0)
    def _():
        for s in range(NSLOT):
            for d in _edata_descs(wid, s):
                d.start()

    def _quad(t, carry):
        quad = t * NW + wid

        for s in range(NSLOT):
            @pl.when(t > 0)
            def _(s=s):
                _scatter_desc(s).wait()
            for d in _edata_descs(quad, s):
                d.wait()
            for j in range(CH // 16):
                sl = pl.ds(j * 16, 16)
                didx[s, sl] = dst_s[s][sl]
            _gather_desc(s).start()

        for s in range(NSLOT):
            _gather_desc(s).wait()

            @pl.when(t + 1 < nq)
            def _(s=s):
                for d in _edata_descs(quad + NW, s):
                    d.start()

            def _scale(g, c2, s=s):
                ew16 = ew_s[s][pl.ds(pl.multiple_of(g * 16, 16), 16)]
                for lane in range(16):
                    e = g * 16 + lane
                    wv = _splat(ew16, lane)
                    for j in range(D // 16):
                        sl = pl.ds(j * 16, 16)
                        rows_s[s][e, sl] = rows_s[s][e, sl] * wv
                return c2

            lax.fori_loop(0, CH // 16, _scale, 0)
            _scatter_desc(s).start(add=True)
        return carry

    lax.fori_loop(0, nq, _quad, 0)

    @pl.when(nq > 0)
    def _():
        for s in range(NSLOT):
            _scatter_desc(s).wait()

    plsc.subcore_barrier()

    for r in range(G_BASE * 8 // DRAIN):
        sl = pl.ds(pl.multiple_of(row0 + r * DRAIN, 8), DRAIN)
        pltpu.sync_copy(acc_sh.at[sl], rows_s[0].at[pl.ds(0, DRAIN)])
        pltpu.sync_copy(rows_s[0].at[pl.ds(0, DRAIN)], out_hbm.at[cid, sl])

    @pl.when(has_extra)
    def _():
        sl = pl.ds(pl.multiple_of(row0 + G_BASE * 8, 8), 8)
        pltpu.sync_copy(acc_sh.at[sl], rows_s[0].at[pl.ds(0, 8)])
        pltpu.sync_copy(rows_s[0].at[pl.ds(0, 8)], out_hbm.at[cid, sl])


def _spmm_sc(h, dst, src, ew):
    mesh = plsc.VectorSubcoreMesh(core_axis_name="c", subcore_axis_name="s")
    f = functools.partial(
        pl.kernel,
        out_type=jax.ShapeDtypeStruct((2, N, D), jnp.float32),
        mesh=mesh,
        scratch_types=[
            [pltpu.VMEM((CH,), jnp.int32) for _ in range(NSLOT)],
            [pltpu.VMEM((CH,), jnp.int32) for _ in range(NSLOT)],
            [pltpu.VMEM((CH,), jnp.float32) for _ in range(NSLOT)],
            [pltpu.VMEM((CH, D), jnp.float32) for _ in range(NSLOT)],
            pltpu.VMEM((NSLOT, CH), jnp.int32),
            pltpu.VMEM_SHARED((N, D), jnp.float32),
            [pltpu.SemaphoreType.DMA for _ in range(NSLOT)],
            [pltpu.SemaphoreType.DMA for _ in range(NSLOT)],
            [pltpu.SemaphoreType.DMA for _ in range(NSLOT)],
        ],
    )(_spmm_body)
    return f(h, dst, src, ew)


def _comb_body(p_ref, o_ref):
    o_ref[...] = p_ref[0] + p_ref[1]


def _combine(parts):
    blk = 1000
    return pl.pallas_call(
        _comb_body,
        grid=(N // blk,),
        in_specs=[pl.BlockSpec((2, blk, D), lambda i: (0, i, 0))],
        out_specs=pl.BlockSpec((blk, D), lambda i: (i, 0)),
        out_shape=jax.ShapeDtypeStruct((N, D), jnp.float32),
    )(parts)


def kernel(x, edge_index, edge_weight, W, b):
    h = _linear(x, W, b)
    parts = _spmm_sc(h, edge_index[0], edge_index[1], edge_weight)
    return _combine(parts)


# speedup vs baseline: 12.4449x; 1.1336x over previous
import functools

import jax
import jax.numpy as jnp
from jax import lax
from jax.experimental import pallas as pl
from jax.experimental.pallas import tpu as pltpu
from jax.experimental.pallas import tpu_sc as plsc

N = 10000
E = 320000
D = 128

CH = 64
NCHUNK = E // CH
NW = 32
NSLOT = 4
NQUAD = NCHUNK // NSLOT
Q_EXTRA = NQUAD % NW
NGROUP = N // 8
G_BASE = NGROUP // 16
G_EXTRA = NGROUP % 16
DRAIN = 48


def _linear_body(x_ref, w_ref, b_ref, o_ref):
    o_ref[...] = lax.dot_general(
        x_ref[...], w_ref[...], (((1,), (1,)), ((), ())),
        preferred_element_type=jnp.float32) + b_ref[...]


def _linear(x, W, b):
    blk = 1000
    return pl.pallas_call(
        _linear_body,
        grid=(N // blk,),
        in_specs=[
            pl.BlockSpec((blk, D), lambda i: (i, 0)),
            pl.BlockSpec((D, D), lambda i: (0, 0)),
            pl.BlockSpec((1, D), lambda i: (0, 0)),
        ],
        out_specs=pl.BlockSpec((blk, D), lambda i: (i, 0)),
        out_shape=jax.ShapeDtypeStruct((N, D), jnp.float32),
    )(x, W, b.reshape(1, D))


_SPLAT_DNUMS = lax.GatherDimensionNumbers(
    offset_dims=(), collapsed_slice_dims=(0,), start_index_map=(0,))


def _splat(vec16, lane):
    idx = jnp.full((16, 1), lane, jnp.int32)
    return lax.gather(vec16, idx, _SPLAT_DNUMS, slice_sizes=(1,),
                      mode=lax.GatherScatterMode.PROMISE_IN_BOUNDS)


def _spmm_body(h_hbm, dst_hbm, src_hbm, ew_hbm, out_hbm,
               dst_s, src_s, ew_s, rows_s, didx, acc_sh,
               esems, gsems, ssems):
    cid = lax.axis_index("c")
    sid = lax.axis_index("s")
    wid = sid * 2 + cid
    row0 = pl.multiple_of(8 * (G_BASE * sid + jnp.minimum(sid, G_EXTRA)), 8)
    has_extra = sid < G_EXTRA

    def _edata_descs(quad, s):
        base = pl.multiple_of((quad * NSLOT + s) * CH, CH)
        return (
            pltpu.make_async_copy(dst_hbm.at[pl.ds(base, CH)], dst_s[s],
                                  esems[s]),
            pltpu.make_async_copy(src_hbm.at[pl.ds(base, CH)], src_s[s],
                                  esems[s]),
            pltpu.make_async_copy(ew_hbm.at[pl.ds(base, CH)], ew_s[s],
                                  esems[s]),
        )

    def _gather_desc(s):
        return pltpu.make_async_copy(h_hbm.at[src_s[s]], rows_s[s], gsems[s])

    def _scatter_desc(s):
        return pltpu.make_async_copy(rows_s[s], acc_sh.at[didx.at[s]],
                                     ssems[s])

    zeros16 = jnp.zeros((16,), jnp.float32)

    def _zrow(i, carry):
        for j in range(D // 16):
            rows_s[0][i, pl.ds(j * 16, 16)] = zeros16
        return carry

    lax.fori_loop(0, DRAIN, _zrow, 0)
    for r in range(G_BASE * 8 // DRAIN):
        pltpu.sync_copy(rows_s[0].at[pl.ds(0, DRAIN)],
                        acc_sh.at[pl.ds(pl.multiple_of(row0 + r * DRAIN, 8),
                                        DRAIN)])

    @pl.when(has_extra)
    def _():
        pltpu.sync_copy(
            rows_s[0].at[pl.ds(0, 8)],
            acc_sh.at[pl.ds(pl.multiple_of(row0 + G_BASE * 8, 8), 8)])

    plsc.subcore_barrier()

    nq = (NQUAD // NW) + jnp.where(wid < Q_EXTRA, 1, 0)

    @pl.when(nq > 0)
    def _():
        for s in range(NSLOT):
            for d in _edata_descs(wid, s):
                d.start()

    def _quad(t, carry):
        quad = t * NW + wid

        for s in range(NSLOT):
            @pl.when(t > 0)
            def _(s=s):
                _scatter_desc(s).wait()
            for d in _edata_descs(quad, s):
                d.wait()
            for j in range(CH // 16):
                sl = pl.ds(j * 16, 16)
                didx[s, sl] = dst_s[s][sl]
            _gather_desc(s).start()

        for s in range(NSLOT):
            _gather_desc(s).wait()

            @pl.when(t + 1 < nq)
            def _(s=s):
                for d in _edata_descs(quad + NW, s):
                    d.start()

            @functools.partial(plsc.parallel_loop, 0, CH // 16, unroll=2)
            def _scale(g, s=s):
                ew16 = ew_s[s][pl.ds(pl.multiple_of(g * 16, 16), 16)]
                for lane in range(16):
                    e = g * 16 + lane
                    wv = _splat(ew16, lane)
                    for j in range(D // 16):
                        sl = pl.ds(j * 16, 16)
                        rows_s[s][e, sl] = rows_s[s][e, sl] * wv
            _scatter_desc(s).start(add=True)
        return carry

    lax.fori_loop(0, nq, _quad, 0)

    @pl.when(nq > 0)
    def _():
        for s in range(NSLOT):
            _scatter_desc(s).wait()

    plsc.subcore_barrier()

    for r in range(G_BASE * 8 // DRAIN):
        sl = pl.ds(pl.multiple_of(row0 + r * DRAIN, 8), DRAIN)
        pltpu.sync_copy(acc_sh.at[sl], rows_s[0].at[pl.ds(0, DRAIN)])
        pltpu.sync_copy(rows_s[0].at[pl.ds(0, DRAIN)], out_hbm.at[cid, sl])

    @pl.when(has_extra)
    def _():
        sl = pl.ds(pl.multiple_of(row0 + G_BASE * 8, 8), 8)
        pltpu.sync_copy(acc_sh.at[sl], rows_s[0].at[pl.ds(0, 8)])
        pltpu.sync_copy(rows_s[0].at[pl.ds(0, 8)], out_hbm.at[cid, sl])


def _spmm_sc(h, dst, src, ew):
    mesh = plsc.VectorSubcoreMesh(core_axis_name="c", subcore_axis_name="s")
    f = functools.partial(
        pl.kernel,
        out_type=jax.ShapeDtypeStruct((2, N, D), jnp.float32),
        mesh=mesh,
        scratch_types=[
            [pltpu.VMEM((CH,), jnp.int32) for _ in range(NSLOT)],
            [pltpu.VMEM((CH,), jnp.int32) for _ in range(NSLOT)],
            [pltpu.VMEM((CH,), jnp.float32) for _ in range(NSLOT)],
            [pltpu.VMEM((CH, D), jnp.float32) for _ in range(NSLOT)],
            pltpu.VMEM((NSLOT, CH), jnp.int32),
            pltpu.VMEM_SHARED((N, D), jnp.float32),
            [pltpu.SemaphoreType.DMA for _ in range(NSLOT)],
            [pltpu.SemaphoreType.DMA for _ in range(NSLOT)],
            [pltpu.SemaphoreType.DMA for _ in range(NSLOT)],
        ],
    )(_spmm_body)
    return f(h, dst, src, ew)


def _comb_body(p_ref, o_ref):
    o_ref[...] = p_ref[0] + p_ref[1]


def _combine(parts):
    blk = 1000
    return pl.pallas_call(
        _comb_body,
        grid=(N // blk,),
        in_specs=[pl.BlockSpec((2, blk, D), lambda i: (0, i, 0))],
        out_specs=pl.BlockSpec((blk, D), lambda i: (i, 0)),
        out_shape=jax.ShapeDtypeStruct((N, D), jnp.float32),
    )(parts)


def kernel(x, edge_index, edge_weight, W, b):
    h = _linear(x, W, b)
    parts = _spmm_sc(h, edge_index[0], edge_index[1], edge_weight)
    return _combine(parts)
